# baseline (speedup 1.0000x reference)
import functools

import jax
import jax.numpy as jnp
from jax import lax
from jax.experimental import pallas as pl
from jax.experimental.pallas import tpu as pltpu
from jax.experimental.pallas import tpu_sc as plsc


_B, _S, _D = 4, 4096, 1024
_E = 128
_NW = 32
_RPW = (_B * _S) // _NW
_CH = 64
_NCH = _RPW // _CH
_SPW = _S // (_NW // _B)


def _sc_body(x_hbm, tab_hbm, out_hbm, x_v, t_v, sem_in, sem_out):
    wid = lax.axis_index("s") * 2 + lax.axis_index("c")
    b = wid // (_NW // _B)
    s0 = (wid % (_NW // _B)) * _SPW

    def chunk(i, _):
        s = s0 + i * _CH
        cin_x = pltpu.make_async_copy(
            x_hbm.at[b, pl.ds(s, _CH), :], x_v, sem_in)
        cin_t = pltpu.make_async_copy(
            tab_hbm.at[pl.ds(s, _CH), :], t_v, sem_in)
        cin_x.start()
        cin_t.start()
        cin_x.wait()
        cin_t.wait()
        cout_x = pltpu.make_async_copy(
            x_v, out_hbm.at[b, pl.ds(s, _CH), pl.ds(0, _D)], sem_out)
        cout_t = pltpu.make_async_copy(
            t_v, out_hbm.at[b, pl.ds(s, _CH), pl.ds(_D, _E)], sem_out)
        cout_x.start()
        cout_t.start()
        cout_x.wait()
        cout_t.wait()
        return 0

    lax.fori_loop(0, _NCH, chunk, 0)


def kernel(x, embed_table):
    b, s, d = x.shape
    e = embed_table.shape[1]
    mesh = plsc.VectorSubcoreMesh(core_axis_name="c", subcore_axis_name="s")
    k = functools.partial(
        pl.kernel,
        mesh=mesh,
        out_type=jax.ShapeDtypeStruct((b, s, d + e), x.dtype),
        scratch_types=[
            pltpu.VMEM((_CH, _D), jnp.float32),
            pltpu.VMEM((_CH, _E), jnp.float32),
            pltpu.SemaphoreType.DMA,
            pltpu.SemaphoreType.DMA,
        ],
    )(_sc_body)
    return k(x, embed_table)


# speedup vs baseline: 1.0330x; 1.0330x over previous
import functools

import jax
import jax.numpy as jnp
from jax import lax
from jax.experimental import pallas as pl
from jax.experimental.pallas import tpu as pltpu
from jax.experimental.pallas import tpu_sc as plsc


_B, _S, _D = 4, 4096, 1024
_E = 128
_NW = 32
_RPW = (_B * _S) // _NW
_CH = 32
_NCH = _RPW // _CH
_SPW = _S // (_NW // _B)


def _sc_body(x_hbm, tab_hbm, out_hbm, x_v, t_v, sem_in, sem_out):
    wid = lax.axis_index("s") * 2 + lax.axis_index("c")
    b = wid // (_NW // _B)
    s0 = (wid % (_NW // _B)) * _SPW

    def in_copies(i, p):
        s = s0 + i * _CH
        return (
            pltpu.make_async_copy(
                x_hbm.at[b, pl.ds(s, _CH), :], x_v.at[p], sem_in.at[p]),
            pltpu.make_async_copy(
                tab_hbm.at[pl.ds(s, _CH), :], t_v.at[p], sem_in.at[p]),
        )

    def out_copies(i, p):
        s = s0 + i * _CH
        return (
            pltpu.make_async_copy(
                x_v.at[p], out_hbm.at[b, pl.ds(s, _CH), pl.ds(0, _D)],
                sem_out.at[p]),
            pltpu.make_async_copy(
                t_v.at[p], out_hbm.at[b, pl.ds(s, _CH), pl.ds(_D, _E)],
                sem_out.at[p]),
        )

    for cp in in_copies(0, 0) + in_copies(1, 1):
        cp.start()
    for i in range(_NCH):
        p = i & 1
        for cp in in_copies(i, p):
            cp.wait()
        outs = out_copies(i, p)
        for cp in outs:
            cp.start()
        if i + 2 < _NCH:
            for cp in outs:
                cp.wait()
            for cp in in_copies(i + 2, p):
                cp.start()
    for i in (_NCH - 2, _NCH - 1):
        for cp in out_copies(i, i & 1):
            cp.wait()


def kernel(x, embed_table):
    b, s, d = x.shape
    e = embed_table.shape[1]
    mesh = plsc.VectorSubcoreMesh(core_axis_name="c", subcore_axis_name="s")
    k = functools.partial(
        pl.kernel,
        mesh=mesh,
        out_type=jax.ShapeDtypeStruct((b, s, d + e), x.dtype),
        scratch_types=[
            pltpu.VMEM((2, _CH, _D), jnp.float32),
            pltpu.VMEM((2, _CH, _E), jnp.float32),
            pltpu.SemaphoreType.DMA((2,)),
            pltpu.SemaphoreType.DMA((2,)),
        ],
    )(_sc_body)
    return k(x, embed_table)


# speedup vs baseline: 1.1094x; 1.0740x over previous
import functools

import jax
import jax.numpy as jnp
from jax import lax
from jax.experimental import pallas as pl
from jax.experimental.pallas import tpu as pltpu
from jax.experimental.pallas import tpu_sc as plsc


_B, _S, _D = 4, 4096, 1024
_E = 128
_NW = 32
_SPW = _S // (_NW // _B)
_SB = 2048


def _sc_body(tab_hbm, out_hbm, t_v, sem_in, sem_out):
    wid = lax.axis_index("s") * 2 + lax.axis_index("c")
    b = wid // (_NW // _B)
    s0 = (wid % (_NW // _B)) * _SPW
    cin = pltpu.make_async_copy(tab_hbm.at[pl.ds(s0, _SPW), :], t_v, sem_in)
    cin.start()
    cin.wait()
    cout = pltpu.make_async_copy(
        t_v, out_hbm.at[b, pl.ds(s0, _SPW), pl.ds(_D, _E)], sem_out)
    cout.start()
    cout.wait()


def _tc_body(alias_ref, x_ref, out_ref):
    del alias_ref
    out_ref[...] = x_ref[...]


def kernel(x, embed_table):
    b, s, d = x.shape
    e = embed_table.shape[1]
    mesh = plsc.VectorSubcoreMesh(core_axis_name="c", subcore_axis_name="s")
    sc_fill = functools.partial(
        pl.kernel,
        mesh=mesh,
        out_type=jax.ShapeDtypeStruct((b, s, d + e), x.dtype),
        scratch_types=[
            pltpu.VMEM((_SPW, _E), jnp.float32),
            pltpu.SemaphoreType.DMA,
            pltpu.SemaphoreType.DMA,
        ],
    )(_sc_body)
    partial_out = sc_fill(embed_table)

    return pl.pallas_call(
        _tc_body,
        grid=(s // _SB, b),
        in_specs=[
            pl.BlockSpec(memory_space=pl.ANY),
            pl.BlockSpec((1, _SB, d), lambda i, j: (j, i, 0)),
        ],
        out_specs=pl.BlockSpec((1, _SB, d), lambda i, j: (j, i, 0)),
        out_shape=jax.ShapeDtypeStruct((b, s, d + e), x.dtype),
        input_output_aliases={0: 0},
        compiler_params=pltpu.CompilerParams(
            dimension_semantics=("parallel", "parallel"),
        ),
    )(partial_out, x)
